# baseline (speedup 1.0000x reference)
import functools

import jax
import jax.numpy as jnp
from jax import lax
from jax.experimental import pallas as pl
from jax.experimental.pallas import tpu as pltpu
from jax.experimental.pallas import tpu_sc as plsc

_N = 1024
_PAD = 128
_CH = 256
_CLS = 19
_KK = 9
_TOPK = 4
_INV_SIGMA2 = 1.0 / (24.0 * 24.0)
_W0, _W1 = 1.0, 0.5
_PAR_DELTA = ((-1, 0, 0), (0, 0, 1))

_NPIX = 8 * _N
_NSUB = 32
_PPS = _NPIX // _NSUB
_L = 16


def _wrap_mask(t_lane, dc):
    if dc == -1:
        return t_lane > 0
    if dc == 1:
        return t_lane < 31
    return None


def _tc_body(lq_ref, f_ref, dens_ref, sim_ref, cpos_ref, cneg_ref, cnt_ref,
             ppad_ref, spad_ref, fpad_ref, simpad_ref, cppad_ref, cnpad_ref):
    npad = _N + 2 * _PAD
    ppad_ref[...] = jnp.zeros((8, _CLS, npad), jnp.float32)
    spad_ref[...] = jnp.zeros((8, npad), jnp.float32)
    fpad_ref[...] = jnp.zeros((2, _CH, npad), jnp.float32)
    cppad_ref[...] = jnp.zeros((40, npad), jnp.float32)
    cnpad_ref[...] = jnp.zeros((40, npad), jnp.float32)
    for r in range(8):
        x = lq_ref[r]
        m = jnp.max(x, axis=0, keepdims=True)
        e = jnp.exp(x - m)
        p = e / jnp.sum(e, axis=0, keepdims=True)
        ppad_ref[r, :, _PAD:_PAD + _N] = p
        spad_ref[r, _PAD:_PAD + _N] = jnp.sum(p, axis=0)
    fpad_ref[:, :, _PAD:_PAD + _N] = f_ref[...]

    n_lane = jax.lax.broadcasted_iota(jnp.int32, (1, _N), 1)
    t_lane = n_lane % 32

    sqn = jnp.zeros((2, _N), jnp.float32)
    for c0 in range(0, _CH, 32):
        fc = f_ref[:, c0:c0 + 32, :]
        sqn = sqn + jnp.sum(fc * fc, axis=1)
    sim_oob = jnp.exp(sqn * (-_INV_SIGMA2))

    sim32 = {(0, 0): jnp.ones((2, _N), jnp.float32)}
    simpad_ref[...] = jnp.zeros((2, _N + 2 * _PAD), jnp.float32)
    for dr, dc in ((0, 1), (1, -1), (1, 0), (1, 1)):
        o = 32 * dr + dc
        wm = _wrap_mask(t_lane, dc)
        d = jnp.zeros((2, _N), jnp.float32)
        for c0 in range(0, _CH, 32):
            fc = f_ref[:, c0:c0 + 32, :]
            fsh = fpad_ref[:, c0:c0 + 32, _PAD + o:_PAD + o + _N]
            if wm is not None:
                fsh = jnp.where(wm[:, None, :], fsh, 0.0)
            d = d + jnp.sum((fsh - fc) ** 2, axis=1)
        sim32[(dr, dc)] = jnp.exp(d * (-_INV_SIGMA2))
        simpad_ref[:, _PAD:_PAD + _N] = sim32[(dr, dc)]
        om = -o
        wmm = _wrap_mask(t_lane, -dc)
        valid = (n_lane + om >= 0) & (n_lane + om < _N)
        if wmm is not None:
            valid = valid & wmm
        ssh = simpad_ref[:, _PAD + om:_PAD + om + _N]
        sim32[(-dr, -dc)] = jnp.where(valid, ssh, sim_oob)

    f0 = f_ref[:, 0, :]
    maskf = (f0 > 0.0).astype(jnp.float32)

    def _geom(ph, pw, i, j):
        dr = _PAR_DELTA[ph][i]
        dc = _PAR_DELTA[pw][j]
        o = 32 * dr + dc
        q2 = ((ph + i + 1) % 2) * 2 + ((pw + j + 1) % 2)
        return dr, dc, o, q2

    dtots = {}
    for ph in (0, 1):
        for pw in (0, 1):
            q = ph * 2 + pw
            pq = ppad_ref[2 * q:2 * q + 2, :, _PAD:_PAD + _N]
            sq = spad_ref[2 * q:2 * q + 2, _PAD:_PAD + _N]
            dtot = jnp.zeros((2, _N), jnp.float32)
            for i in range(3):
                for j in range(3):
                    k = i * 3 + j
                    if k > 4:
                        continue
                    dr, dc, o, q2 = _geom(ph, pw, i, j)
                    psh = ppad_ref[2 * q2:2 * q2 + 2, :, _PAD + o:_PAD + o + _N]
                    ssh = spad_ref[2 * q2:2 * q2 + 2, _PAD + o:_PAD + o + _N]
                    wm = _wrap_mask(t_lane, dc)
                    if wm is not None:
                        psh = jnp.where(wm[:, None, :], psh, 0.0)
                        ssh = jnp.where(wm, ssh, 0.0)
                    cp = jnp.sum(pq * psh, axis=1)
                    cn = sq * ssh - cp
                    row = 2 * (q * 5 + k)
                    cppad_ref[row:row + 2, _PAD:_PAD + _N] = cp
                    cnpad_ref[row:row + 2, _PAD:_PAD + _N] = cn
                    sk = sim32[(dr, dc)]
                    dtot = dtot + sk
                    cpm = cp * maskf
                    cnm = cn * maskf
                    for b in (0, 1):
                        off = k * _NPIX + (2 * q + b) * _N
                        sim_ref[pl.ds(off, _N)] = sk[b]
                        cpos_ref[pl.ds(off, _N)] = cpm[b]
                        cneg_ref[pl.ds(off, _N)] = cnm[b]
            dtots[q] = dtot
    for ph in (0, 1):
        for pw in (0, 1):
            q = ph * 2 + pw
            dtot = dtots[q]
            for i in range(3):
                for j in range(3):
                    k = i * 3 + j
                    if k <= 4:
                        continue
                    dr, dc, o, q2 = _geom(ph, pw, i, j)
                    row = 2 * (q2 * 5 + (8 - k))
                    valid = (n_lane + o >= 0) & (n_lane + o < _N)
                    wm = _wrap_mask(t_lane, dc)
                    if wm is not None:
                        valid = valid & wm
                    cp = jnp.where(
                        valid, cppad_ref[row:row + 2, _PAD + o:_PAD + o + _N],
                        0.0)
                    cn = jnp.where(
                        valid, cnpad_ref[row:row + 2, _PAD + o:_PAD + o + _N],
                        0.0)
                    sk = sim32[(dr, dc)]
                    dtot = dtot + sk
                    cpm = cp * maskf
                    cnm = cn * maskf
                    for b in (0, 1):
                        off = k * _NPIX + (2 * q + b) * _N
                        sim_ref[pl.ds(off, _N)] = sk[b]
                        cpos_ref[pl.ds(off, _N)] = cpm[b]
                        cneg_ref[pl.ds(off, _N)] = cnm[b]
            dtots[q] = 1.0 - dtot * (1.0 / _KK)
    d00 = dtots[0].reshape(2, 32, 32)
    d01 = dtots[1].reshape(2, 32, 32)
    d10 = dtots[2].reshape(2, 32, 32)
    d11 = dtots[3].reshape(2, 32, 32)
    r0 = jnp.stack((d00, d01), axis=-1).reshape(2, 32, 64)
    r1 = jnp.stack((d10, d11), axis=-1).reshape(2, 32, 64)
    dens_ref[:, 0, :, :] = jnp.stack((r0, r1), axis=2).reshape(2, 64, 64)
    cnt_ref[...] = jnp.full((1, 1), jnp.sum(maskf) * 4.0, jnp.float32)


def _tc_call(lq, feats):
    npad = _N + 2 * _PAD
    return pl.pallas_call(
        _tc_body,
        out_shape=[
            jax.ShapeDtypeStruct((2, 1, 64, 64), jnp.float32),
            jax.ShapeDtypeStruct((_KK * _NPIX,), jnp.float32),
            jax.ShapeDtypeStruct((_KK * _NPIX,), jnp.float32),
            jax.ShapeDtypeStruct((_KK * _NPIX,), jnp.float32),
            jax.ShapeDtypeStruct((1, 1), jnp.float32),
        ],
        scratch_shapes=[
            pltpu.VMEM((8, _CLS, npad), jnp.float32),
            pltpu.VMEM((8, npad), jnp.float32),
            pltpu.VMEM((2, _CH, npad), jnp.float32),
            pltpu.VMEM((2, npad), jnp.float32),
            pltpu.VMEM((40, npad), jnp.float32),
            pltpu.VMEM((40, npad), jnp.float32),
        ],
    )(lq, feats)


@functools.cache
def _get_sc_select():
    return functools.partial(
        pl.kernel,
        out_type=jax.ShapeDtypeStruct((_NSUB * 2 * _L,), jnp.float32),
        mesh=plsc.VectorSubcoreMesh(core_axis_name="c", subcore_axis_name="s",
                                    num_cores=2, num_subcores=16),
        scratch_types=[
            pltpu.VMEM((_KK * _PPS,), jnp.float32),
            pltpu.VMEM((_KK * _PPS,), jnp.float32),
            pltpu.VMEM((_KK * _PPS,), jnp.float32),
            pltpu.VMEM((2 * _L,), jnp.float32),
            pltpu.SemaphoreType.DMA,
        ],
    )(_sc_select_body)


def _sc_select_body(sim_hbm, cpos_hbm, cneg_hbm, out_hbm,
                    sim_v, cpos_v, cneg_v, out_v, sem):
    wid = lax.axis_index("s") * 2 + lax.axis_index("c")
    base = wid * _PPS
    copies = []
    for k in range(_KK):
        for hbm, vmem in ((sim_hbm, sim_v), (cpos_hbm, cpos_v),
                          (cneg_hbm, cneg_v)):
            copies.append(pltpu.async_copy(
                hbm.at[pl.ds(k * _NPIX + base, _PPS)],
                vmem.at[pl.ds(k * _PPS, _PPS)], sem))
    for cp in copies:
        cp.wait()

    acc_p = jnp.zeros((_L,), jnp.float32)
    acc_n = jnp.zeros((_L,), jnp.float32)
    for c in range(_PPS // _L):
        s = [sim_v[pl.ds(k * _PPS + c * _L, _L)] for k in range(_KK)]
        rmax = [jnp.full((_L,), k, jnp.int32) for k in range(_KK)]
        rmin = [jnp.full((_L,), k, jnp.int32) for k in range(_KK)]
        for a in range(_KK):
            for b in range(a + 1, _KK):
                lt = jnp.where(s[a] < s[b], 1, 0)
                gt = jnp.where(s[a] > s[b], 1, 0)
                rmax[b] = rmax[b] - lt
                rmax[a] = rmax[a] + lt
                rmin[b] = rmin[b] - gt
                rmin[a] = rmin[a] + gt
        chunk_p = jnp.zeros((_L,), jnp.float32)
        chunk_n = jnp.zeros((_L,), jnp.float32)
        for k in range(_KK):
            ksl = pl.ds(k * _PPS + c * _L, _L)
            chunk_p = chunk_p + jnp.where(rmax[k] < _TOPK + 1,
                                          s[k] * cpos_v[ksl], 0.0)
            chunk_n = chunk_n + jnp.where(rmin[k] < _TOPK,
                                          (1.0 - s[k]) * cneg_v[ksl], 0.0)
        acc_p = acc_p + chunk_p
        acc_n = acc_n + chunk_n

    out_v[pl.ds(0, _L)] = acc_p
    out_v[pl.ds(_L, _L)] = acc_n
    pltpu.sync_copy(out_v, out_hbm.at[pl.ds(wid * 2 * _L, 2 * _L)])


def kernel(logits_trg, x_trg_2, x_ema_2, img_trg):
    B, C, H, W = logits_trg.shape
    del x_trg_2, img_trg
    lq = jnp.stack([logits_trg[:, :, ph::2, pw::2]
                    for ph in (0, 1) for pw in (0, 1)])
    lq = lq.reshape(4 * B, C, _N)
    feats = x_ema_2.reshape(B, _CH, _N)
    dens, sim, cpos, cneg, cnt11 = _tc_call(lq, feats)
    partials = _get_sc_select()(sim, cpos, cneg)
    partials = partials.reshape(_NSUB, 2, _L)
    ps = jnp.sum(partials[:, 0, :])
    ns = jnp.sum(partials[:, 1, :])
    cnt = cnt11[0, 0]
    loss_pos = -ps / (cnt * (_TOPK + 1)) * _W0
    loss_neg = -ns / (cnt * _TOPK) * _W1
    return loss_pos, loss_neg, dens


# speedup vs baseline: 1.1683x; 1.1683x over previous
import functools

import jax
import jax.numpy as jnp
from jax import lax
from jax.experimental import pallas as pl
from jax.experimental.pallas import tpu as pltpu
from jax.experimental.pallas import tpu_sc as plsc

_N = 1024
_PAD = 128
_CH = 256
_CLS = 19
_KK = 9
_TOPK = 4
_INV_SIGMA2 = 1.0 / (24.0 * 24.0)
_W0, _W1 = 1.0, 0.5
_PAR_DELTA = ((-1, 0, 0), (0, 0, 1))

_NPIX = 8 * _N
_NSUB = 32
_PPS = _NPIX // _NSUB
_L = 16


def _wrap_mask(t_lane, dc):
    if dc == -1:
        return t_lane > 0
    if dc == 1:
        return t_lane < 31
    return None


def _tc_body(lq_ref, f_ref, dens_ref, maps_ref, cnt_ref,
             ppad_ref, spad_ref, fpad_ref, simpad_ref, cppad_ref, cnpad_ref):
    npad = _N + 2 * _PAD
    ppad_ref[...] = jnp.zeros((8, _CLS, npad), jnp.float32)
    spad_ref[...] = jnp.zeros((8, npad), jnp.float32)
    fpad_ref[...] = jnp.zeros((2, _CH, npad), jnp.float32)
    cppad_ref[...] = jnp.zeros((40, npad), jnp.float32)
    cnpad_ref[...] = jnp.zeros((40, npad), jnp.float32)
    for r in range(8):
        x = lq_ref[r]
        m = jnp.max(x, axis=0, keepdims=True)
        e = jnp.exp(x - m)
        p = e / jnp.sum(e, axis=0, keepdims=True)
        ppad_ref[r, :, _PAD:_PAD + _N] = p
        spad_ref[r, _PAD:_PAD + _N] = jnp.sum(p, axis=0)
    fpad_ref[:, :, _PAD:_PAD + _N] = f_ref[...]

    n_lane = jax.lax.broadcasted_iota(jnp.int32, (1, _N), 1)
    t_lane = n_lane % 32

    sqn = jnp.zeros((2, _N), jnp.float32)
    for c0 in range(0, _CH, 32):
        fc = f_ref[:, c0:c0 + 32, :]
        sqn = sqn + jnp.sum(fc * fc, axis=1)
    sim_oob = jnp.exp(sqn * (-_INV_SIGMA2))

    sim32 = {(0, 0): jnp.ones((2, _N), jnp.float32)}
    simpad_ref[...] = jnp.zeros((2, _N + 2 * _PAD), jnp.float32)
    for dr, dc in ((0, 1), (1, -1), (1, 0), (1, 1)):
        o = 32 * dr + dc
        wm = _wrap_mask(t_lane, dc)
        d = jnp.zeros((2, _N), jnp.float32)
        for c0 in range(0, _CH, 32):
            fc = f_ref[:, c0:c0 + 32, :]
            fsh = fpad_ref[:, c0:c0 + 32, _PAD + o:_PAD + o + _N]
            if wm is not None:
                fsh = jnp.where(wm[:, None, :], fsh, 0.0)
            d = d + jnp.sum((fsh - fc) ** 2, axis=1)
        sim32[(dr, dc)] = jnp.exp(d * (-_INV_SIGMA2))
        simpad_ref[:, _PAD:_PAD + _N] = sim32[(dr, dc)]
        om = -o
        wmm = _wrap_mask(t_lane, -dc)
        valid = (n_lane + om >= 0) & (n_lane + om < _N)
        if wmm is not None:
            valid = valid & wmm
        ssh = simpad_ref[:, _PAD + om:_PAD + om + _N]
        sim32[(-dr, -dc)] = jnp.where(valid, ssh, sim_oob)

    f0 = f_ref[:, 0, :]
    maskf = (f0 > 0.0).astype(jnp.float32)

    def _geom(ph, pw, i, j):
        dr = _PAR_DELTA[ph][i]
        dc = _PAR_DELTA[pw][j]
        o = 32 * dr + dc
        q2 = ((ph + i + 1) % 2) * 2 + ((pw + j + 1) % 2)
        return dr, dc, o, q2

    dtots = {}
    for ph in (0, 1):
        for pw in (0, 1):
            q = ph * 2 + pw
            pq = ppad_ref[2 * q:2 * q + 2, :, _PAD:_PAD + _N]
            sq = spad_ref[2 * q:2 * q + 2, _PAD:_PAD + _N]
            dtot = jnp.zeros((2, _N), jnp.float32)
            for i in range(3):
                for j in range(3):
                    k = i * 3 + j
                    if k > 4:
                        continue
                    dr, dc, o, q2 = _geom(ph, pw, i, j)
                    psh = ppad_ref[2 * q2:2 * q2 + 2, :, _PAD + o:_PAD + o + _N]
                    ssh = spad_ref[2 * q2:2 * q2 + 2, _PAD + o:_PAD + o + _N]
                    wm = _wrap_mask(t_lane, dc)
                    if wm is not None:
                        psh = jnp.where(wm[:, None, :], psh, 0.0)
                        ssh = jnp.where(wm, ssh, 0.0)
                    cp = jnp.sum(pq * psh, axis=1)
                    cn = sq * ssh - cp
                    row = 2 * (q * 5 + k)
                    cppad_ref[row:row + 2, _PAD:_PAD + _N] = cp
                    cnpad_ref[row:row + 2, _PAD:_PAD + _N] = cn
                    sk = sim32[(dr, dc)]
                    dtot = dtot + sk
                    cpm = cp * maskf
                    cnm = cn * maskf
                    for b in (0, 1):
                        for jc in range(4):
                            wid = (2 * q + b) * 4 + jc
                            dst = wid * 27 * _PPS + 3 * k * _PPS
                            sl = slice(jc * _PPS, (jc + 1) * _PPS)
                            maps_ref[pl.ds(dst, _PPS)] = sk[b][sl]
                            maps_ref[pl.ds(dst + _PPS, _PPS)] = cpm[b][sl]
                            maps_ref[pl.ds(dst + 2 * _PPS, _PPS)] = cnm[b][sl]
            dtots[q] = dtot
    for ph in (0, 1):
        for pw in (0, 1):
            q = ph * 2 + pw
            dtot = dtots[q]
            for i in range(3):
                for j in range(3):
                    k = i * 3 + j
                    if k <= 4:
                        continue
                    dr, dc, o, q2 = _geom(ph, pw, i, j)
                    row = 2 * (q2 * 5 + (8 - k))
                    valid = (n_lane + o >= 0) & (n_lane + o < _N)
                    wm = _wrap_mask(t_lane, dc)
                    if wm is not None:
                        valid = valid & wm
                    cp = jnp.where(
                        valid, cppad_ref[row:row + 2, _PAD + o:_PAD + o + _N],
                        0.0)
                    cn = jnp.where(
                        valid, cnpad_ref[row:row + 2, _PAD + o:_PAD + o + _N],
                        0.0)
                    sk = sim32[(dr, dc)]
                    dtot = dtot + sk
                    cpm = cp * maskf
                    cnm = cn * maskf
                    for b in (0, 1):
                        for jc in range(4):
                            wid = (2 * q + b) * 4 + jc
                            dst = wid * 27 * _PPS + 3 * k * _PPS
                            sl = slice(jc * _PPS, (jc + 1) * _PPS)
                            maps_ref[pl.ds(dst, _PPS)] = sk[b][sl]
                            maps_ref[pl.ds(dst + _PPS, _PPS)] = cpm[b][sl]
                            maps_ref[pl.ds(dst + 2 * _PPS, _PPS)] = cnm[b][sl]
            dens_ref[2 * q:2 * q + 2, :] = 1.0 - dtot * (1.0 / _KK)
    cnt_ref[...] = jnp.full((1, 1), jnp.sum(maskf) * 4.0, jnp.float32)


def _tc_call(lq, feats):
    npad = _N + 2 * _PAD
    return pl.pallas_call(
        _tc_body,
        out_shape=[
            jax.ShapeDtypeStruct((8, _N), jnp.float32),
            jax.ShapeDtypeStruct((_NSUB * 27 * _PPS,), jnp.float32),
            jax.ShapeDtypeStruct((1, 1), jnp.float32),
        ],
        scratch_shapes=[
            pltpu.VMEM((8, _CLS, npad), jnp.float32),
            pltpu.VMEM((8, npad), jnp.float32),
            pltpu.VMEM((2, _CH, npad), jnp.float32),
            pltpu.VMEM((2, npad), jnp.float32),
            pltpu.VMEM((40, npad), jnp.float32),
            pltpu.VMEM((40, npad), jnp.float32),
        ],
    )(lq, feats)


@functools.cache
def _get_sc_select():
    return functools.partial(
        pl.kernel,
        out_type=jax.ShapeDtypeStruct((_NSUB * 2 * _L,), jnp.float32),
        mesh=plsc.VectorSubcoreMesh(core_axis_name="c", subcore_axis_name="s",
                                    num_cores=2, num_subcores=16),
        scratch_types=[
            pltpu.VMEM((27 * _PPS,), jnp.float32),
            pltpu.VMEM((2 * _L,), jnp.float32),
            pltpu.SemaphoreType.DMA,
        ],
    )(_sc_select_body)


def _sc_select_body(maps_hbm, out_hbm, maps_v, out_v, sem):
    wid = lax.axis_index("s") * 2 + lax.axis_index("c")
    pltpu.async_copy(maps_hbm.at[pl.ds(wid * 27 * _PPS, 27 * _PPS)],
                     maps_v, sem).wait()

    acc_p = jnp.zeros((_L,), jnp.float32)
    acc_n = jnp.zeros((_L,), jnp.float32)
    for c in range(_PPS // _L):
        s = [maps_v[pl.ds(3 * k * _PPS + c * _L, _L)] for k in range(_KK)]
        rmax = [jnp.full((_L,), k, jnp.int32) for k in range(_KK)]
        rmin = [jnp.full((_L,), k, jnp.int32) for k in range(_KK)]
        for a in range(_KK):
            for b in range(a + 1, _KK):
                lt = jnp.where(s[a] < s[b], 1, 0)
                gt = jnp.where(s[a] > s[b], 1, 0)
                rmax[b] = rmax[b] - lt
                rmax[a] = rmax[a] + lt
                rmin[b] = rmin[b] - gt
                rmin[a] = rmin[a] + gt
        chunk_p = jnp.zeros((_L,), jnp.float32)
        chunk_n = jnp.zeros((_L,), jnp.float32)
        for k in range(_KK):
            cpsl = pl.ds((3 * k + 1) * _PPS + c * _L, _L)
            cnsl = pl.ds((3 * k + 2) * _PPS + c * _L, _L)
            chunk_p = chunk_p + jnp.where(rmax[k] < _TOPK + 1,
                                          s[k] * maps_v[cpsl], 0.0)
            chunk_n = chunk_n + jnp.where(rmin[k] < _TOPK,
                                          (1.0 - s[k]) * maps_v[cnsl], 0.0)
        acc_p = acc_p + chunk_p
        acc_n = acc_n + chunk_n

    out_v[pl.ds(0, _L)] = acc_p
    out_v[pl.ds(_L, _L)] = acc_n
    pltpu.sync_copy(out_v, out_hbm.at[pl.ds(wid * 2 * _L, 2 * _L)])


def kernel(logits_trg, x_trg_2, x_ema_2, img_trg):
    B, C, H, W = logits_trg.shape
    del x_trg_2, img_trg
    lq = jnp.stack([logits_trg[:, :, ph::2, pw::2]
                    for ph in (0, 1) for pw in (0, 1)])
    lq = lq.reshape(4 * B, C, _N)
    feats = x_ema_2.reshape(B, _CH, _N)
    dens, maps, cnt11 = _tc_call(lq, feats)
    partials = _get_sc_select()(maps)
    partials = partials.reshape(_NSUB, 2, _L)
    ps = jnp.sum(partials[:, 0, :])
    ns = jnp.sum(partials[:, 1, :])
    cnt = cnt11[0, 0]
    loss_pos = -ps / (cnt * (_TOPK + 1)) * _W0
    loss_neg = -ns / (cnt * _TOPK) * _W1
    density = (dens.reshape(2, 2, B, 32, 32)
               .transpose(2, 3, 0, 4, 1)
               .reshape(B, 1, H, W))
    return loss_pos, loss_neg, density
